# baseline (speedup 1.0000x reference)
import functools

import jax
import jax.numpy as jnp
from jax import lax
from jax.experimental import pallas as pl
from jax.experimental.pallas import tpu as pltpu
from jax.experimental.pallas import tpu_sc as plsc

N_BINS = 10
BLK = 128


def _make_partials(N, C):
    info = plsc.get_sparse_core_info()
    NCORES, NSUB, L = info.num_cores, info.num_subcores, info.num_lanes
    W = NCORES * NSUB
    assert N % BLK == 0 and C == 16 and L == 16
    SB = N // BLK
    base = SB // W
    rem = SB % W
    NB = 16
    n_full = base // NB
    NB_TAIL = base - n_full * NB
    GPB = BLK // L
    BW = 8 * BLK
    ACC = 3 * N_BINS * L
    HALF = SB * BW
    assert n_full % 2 == 0

    mesh = plsc.VectorSubcoreMesh(core_axis_name="c", subcore_axis_name="s")

    @functools.partial(
        pl.kernel,
        mesh=mesh,
        out_type=jax.ShapeDtypeStruct((W, ACC), jnp.float32),
        scratch_types=[
            pltpu.VMEM((2 * NB * BW,), jnp.float32),
            pltpu.VMEM((2 * NB * BW,), jnp.float32),
            pltpu.VMEM((NB * BLK,), jnp.int32),
            pltpu.VMEM((NB * BLK,), jnp.int32),
            pltpu.VMEM((ACC,), jnp.float32),
            pltpu.SemaphoreType.DMA,
            pltpu.SemaphoreType.DMA,
        ],
        compiler_params=pltpu.CompilerParams(needs_layout_passes=False),
    )
    def ece_k(flat_hbm, labels_hbm, out_hbm, buf0, buf1, lbuf0, lbuf1,
              acc, sem0, sem1):
        bufs, lbufs, sems = [buf0, buf1], [lbuf0, lbuf1], [sem0, sem1]
        wid = lax.axis_index("s") * NCORES + lax.axis_index("c")
        lane = lax.iota(jnp.int32, L)
        ones = jnp.ones((L,), jnp.float32)
        for k in range(3 * N_BINS):
            acc[pl.ds(k * L, L)] = jnp.zeros((L,), jnp.float32)

        b0 = base * wid + jnp.minimum(wid, rem)

        def do_group(buf, lbuf, boff, goff, t, halfw):
            off = boff + t * L
            vs = ([buf[pl.ds(off + c * BLK, L)] for c in range(8)]
                  + [buf[pl.ds(halfw + off + c * BLK, L)] for c in range(8)])
            lab = lbuf[pl.ds(goff + t * L, L)]
            labidx = (off + lane) + (lab >> 3) * halfw + (lab & 7) * BLK
            vlab = plsc.load_gather(buf, [labidx])
            while len(vs) > 1:
                vs = [jnp.maximum(vs[i], vs[i + 1])
                      for i in range(0, len(vs) - 1, 2)] + (
                          [vs[-1]] if len(vs) % 2 else [])
            m = vs[0]
            conf = 1.0 / (1.0 + jnp.exp(-m))
            accv = jnp.where(vlab == m, 1.0, 0.0).astype(jnp.float32)
            bi = jnp.minimum((conf * N_BINS).astype(jnp.int32), N_BINS - 1)
            fidx = bi * L + lane
            plsc.addupdate_scatter(acc, [fidx], ones)
            plsc.addupdate_scatter(acc, [fidx + (N_BINS * L)], conf)
            plsc.addupdate_scatter(acc, [fidx + (2 * N_BINS * L)], accv)

        def start(b, nb, slot):
            h0 = pltpu.async_copy(
                flat_hbm.at[pl.ds(b * BW, nb * BW)],
                bufs[slot].at[pl.ds(0, nb * BW)], sems[slot])
            h1 = pltpu.async_copy(
                flat_hbm.at[pl.ds(HALF + b * BW, nb * BW)],
                bufs[slot].at[pl.ds(nb * BW, nb * BW)], sems[slot])
            h2 = pltpu.async_copy(
                labels_hbm.at[pl.ds(b * BLK, nb * BLK)],
                lbufs[slot].at[pl.ds(0, nb * BLK)], sems[slot])
            return h0, h1, h2

        def drain(b, nb, slot):
            pltpu.make_async_copy(
                flat_hbm.at[pl.ds(b * BW, nb * BW)],
                bufs[slot].at[pl.ds(0, nb * BW)], sems[slot]).wait()
            pltpu.make_async_copy(
                flat_hbm.at[pl.ds(HALF + b * BW, nb * BW)],
                bufs[slot].at[pl.ds(nb * BW, nb * BW)], sems[slot]).wait()
            pltpu.make_async_copy(
                labels_hbm.at[pl.ds(b * BLK, nb * BLK)],
                lbufs[slot].at[pl.ds(0, nb * BLK)], sems[slot]).wait()

        def compute_chunk(slot, nb):
            halfw = nb * BW

            def blk_body(jb, carry):
                boff = jb * BW
                goff = jb * BLK
                for t in range(GPB):
                    do_group(bufs[slot], lbufs[slot], boff, goff, t, halfw)
                return carry

            lax.fori_loop(0, nb, blk_body, 0)

        start(b0, NB, 0)
        start(b0 + NB, NB, 1)

        def pair_body(j, carry):
            c0 = 2 * j
            bb = b0 + c0 * NB
            drain(bb, NB, 0)
            compute_chunk(0, NB)

            @pl.when(c0 + 2 < n_full)
            def _():
                start(bb + 2 * NB, NB, 0)

            drain(bb + NB, NB, 1)
            compute_chunk(1, NB)

            @pl.when(c0 + 3 < n_full)
            def _():
                start(bb + 3 * NB, NB, 1)

            return carry

        lax.fori_loop(0, n_full // 2, pair_body, 0)

        if NB_TAIL:
            bt = b0 + n_full * NB
            start(bt, NB_TAIL, 0)
            drain(bt, NB_TAIL, 0)
            compute_chunk(0, NB_TAIL)

        @pl.when(wid < rem)
        def _():
            bx = b0 + base
            start(bx, 1, 1)
            drain(bx, 1, 1)
            compute_chunk(1, 1)

        pltpu.sync_copy(acc, out_hbm.at[wid])

    return ece_k, W, L


def kernel(logits, labels):
    N, C = logits.shape
    part_fn, W, L = _make_partials(N, C)
    flat = logits.T.reshape(2, 8, N // BLK, BLK).transpose(0, 2, 1, 3).reshape(-1)
    parts = part_fn(flat, labels.astype(jnp.int32))
    tot = parts.sum(axis=0).reshape(3, N_BINS, L).sum(axis=-1)
    count, sconf, sacc = tot[0], tot[1], tot[2]
    prop = count / N
    safe = jnp.maximum(count, 1.0)
    diff = jnp.abs(sconf / safe - sacc / safe)
    ece = jnp.sum(jnp.where(count > 0, diff * prop, 0.0), dtype=jnp.float32)
    return ece.reshape(1)


# speedup vs baseline: 2.0070x; 2.0070x over previous
import functools

import jax
import jax.numpy as jnp
from jax import lax
from jax.experimental import pallas as pl
from jax.experimental.pallas import tpu as pltpu
from jax.experimental.pallas import tpu_sc as plsc

N_BINS = 10
BLK = 128


def _make_partials(N, C):
    info = plsc.get_sparse_core_info()
    NCORES, NSUB, L = info.num_cores, info.num_subcores, info.num_lanes
    W = NCORES * NSUB
    assert N % BLK == 0 and C == 16 and L == 16
    SB = N // BLK
    base = SB // W
    rem = SB % W
    NB = 16
    n_full = base // NB
    NB_TAIL = base - n_full * NB
    GPB = BLK // L
    BW = 8 * BLK
    ACC = 3 * N_BINS * L
    HALF = SB * BW
    assert n_full % 2 == 0

    mesh = plsc.VectorSubcoreMesh(core_axis_name="c", subcore_axis_name="s")

    @functools.partial(
        pl.kernel,
        mesh=mesh,
        out_type=jax.ShapeDtypeStruct((W, ACC), jnp.float32),
        scratch_types=[
            pltpu.VMEM((2 * NB * BW,), jnp.float32),
            pltpu.VMEM((2 * NB * BW,), jnp.float32),
            pltpu.VMEM((NB * BLK,), jnp.int32),
            pltpu.VMEM((NB * BLK,), jnp.int32),
            pltpu.VMEM((ACC,), jnp.float32),
            pltpu.SemaphoreType.DMA,
            pltpu.SemaphoreType.DMA,
        ],
        compiler_params=pltpu.CompilerParams(needs_layout_passes=False),
    )
    def ece_k(flat_hbm, labels_hbm, out_hbm, buf0, buf1, lbuf0, lbuf1,
              acc, sem0, sem1):
        bufs, lbufs, sems = [buf0, buf1], [lbuf0, lbuf1], [sem0, sem1]
        wid = lax.axis_index("s") * NCORES + lax.axis_index("c")
        lane = lax.iota(jnp.int32, L)
        ones = jnp.ones((L,), jnp.float32)
        for k in range(3 * N_BINS):
            acc[pl.ds(k * L, L)] = jnp.zeros((L,), jnp.float32)

        b0 = base * wid + jnp.minimum(wid, rem)

        def block_max(buf, boff, halfw):
            ms = []
            for t in range(GPB):
                off = boff + t * L
                vs = ([buf[pl.ds(off + c * BLK, L)] for c in range(8)]
                      + [buf[pl.ds(halfw + off + c * BLK, L)]
                         for c in range(8)])
                while len(vs) > 1:
                    vs = [jnp.maximum(vs[i], vs[i + 1])
                          for i in range(0, len(vs) - 1, 2)]
                ms.append(vs[0])
            return tuple(ms)

        def block_finish(buf, lbuf, boff, goff, halfw, ms):
            exps = [jnp.exp(-m) for m in ms]
            confs = [1.0 / (1.0 + e) for e in exps]
            labs = [lbuf[pl.ds(goff + t * L, L)] for t in range(GPB)]
            vlabs = []
            for t in range(GPB):
                labidx = ((boff + t * L + lane) + (labs[t] >> 3) * halfw
                          + (labs[t] & 7) * BLK)
                vlabs.append(plsc.load_gather(buf, [labidx]))
            for t in range(GPB):
                accv = jnp.where(vlabs[t] == ms[t], 1.0, 0.0)
                accv = accv.astype(jnp.float32)
                bi = jnp.minimum((confs[t] * N_BINS).astype(jnp.int32),
                                 N_BINS - 1)
                fidx = bi * L + lane
                plsc.addupdate_scatter(acc, [fidx], ones)
                plsc.addupdate_scatter(acc, [fidx + (N_BINS * L)], confs[t])
                plsc.addupdate_scatter(acc, [fidx + (2 * N_BINS * L)], accv)

        def start(b, nb, slot):
            h0 = pltpu.async_copy(
                flat_hbm.at[pl.ds(b * BW, nb * BW)],
                bufs[slot].at[pl.ds(0, nb * BW)], sems[slot])
            h1 = pltpu.async_copy(
                flat_hbm.at[pl.ds(HALF + b * BW, nb * BW)],
                bufs[slot].at[pl.ds(nb * BW, nb * BW)], sems[slot])
            h2 = pltpu.async_copy(
                labels_hbm.at[pl.ds(b * BLK, nb * BLK)],
                lbufs[slot].at[pl.ds(0, nb * BLK)], sems[slot])
            return h0, h1, h2

        def drain(b, nb, slot):
            pltpu.make_async_copy(
                flat_hbm.at[pl.ds(b * BW, nb * BW)],
                bufs[slot].at[pl.ds(0, nb * BW)], sems[slot]).wait()
            pltpu.make_async_copy(
                flat_hbm.at[pl.ds(HALF + b * BW, nb * BW)],
                bufs[slot].at[pl.ds(nb * BW, nb * BW)], sems[slot]).wait()
            pltpu.make_async_copy(
                labels_hbm.at[pl.ds(b * BLK, nb * BLK)],
                lbufs[slot].at[pl.ds(0, nb * BLK)], sems[slot]).wait()

        def compute_chunk(slot, nb):
            halfw = nb * BW
            buf, lbuf = bufs[slot], lbufs[slot]

            def blk_body(jb, ms_prev):
                ms_new = block_max(buf, jb * BW, halfw)
                block_finish(buf, lbuf, (jb - 1) * BW, (jb - 1) * BLK,
                             halfw, ms_prev)
                return ms_new

            ms0 = block_max(buf, 0, halfw)
            if nb > 1:
                ms_last = lax.fori_loop(1, nb, blk_body, ms0)
            else:
                ms_last = ms0
            block_finish(buf, lbuf, (nb - 1) * BW, (nb - 1) * BLK,
                         halfw, ms_last)

        start(b0, NB, 0)
        start(b0 + NB, NB, 1)

        def pair_body(j, carry):
            c0 = 2 * j
            bb = b0 + c0 * NB
            drain(bb, NB, 0)
            compute_chunk(0, NB)

            @pl.when(c0 + 2 < n_full)
            def _():
                start(bb + 2 * NB, NB, 0)

            drain(bb + NB, NB, 1)
            compute_chunk(1, NB)

            @pl.when(c0 + 3 < n_full)
            def _():
                start(bb + 3 * NB, NB, 1)

            return carry

        lax.fori_loop(0, n_full // 2, pair_body, 0)

        if NB_TAIL:
            bt = b0 + n_full * NB
            start(bt, NB_TAIL, 0)
            drain(bt, NB_TAIL, 0)
            compute_chunk(0, NB_TAIL)

        @pl.when(wid < rem)
        def _():
            bx = b0 + base
            start(bx, 1, 1)
            drain(bx, 1, 1)
            compute_chunk(1, 1)

        pltpu.sync_copy(acc, out_hbm.at[wid])

    return ece_k, W, L


def kernel(logits, labels):
    N, C = logits.shape
    part_fn, W, L = _make_partials(N, C)
    flat = logits.T.reshape(2, 8, N // BLK, BLK).transpose(0, 2, 1, 3).reshape(-1)
    parts = part_fn(flat, labels.astype(jnp.int32))
    tot = parts.sum(axis=0).reshape(3, N_BINS, L).sum(axis=-1)
    count, sconf, sacc = tot[0], tot[1], tot[2]
    prop = count / N
    safe = jnp.maximum(count, 1.0)
    diff = jnp.abs(sconf / safe - sacc / safe)
    ece = jnp.sum(jnp.where(count > 0, diff * prop, 0.0), dtype=jnp.float32)
    return ece.reshape(1)


# speedup vs baseline: 2.0730x; 1.0329x over previous
import functools

import jax
import jax.numpy as jnp
from jax import lax
from jax.experimental import pallas as pl
from jax.experimental.pallas import tpu as pltpu
from jax.experimental.pallas import tpu_sc as plsc

N_BINS = 10
BLK = 128


def _make_partials(N, C):
    info = plsc.get_sparse_core_info()
    NCORES, NSUB, L = info.num_cores, info.num_subcores, info.num_lanes
    W = NCORES * NSUB
    assert N % BLK == 0 and C == 16 and L == 16
    SB = N // BLK
    base = SB // W
    rem = SB % W
    NB = 16
    n_full = base // NB
    NB_TAIL = base - n_full * NB
    GPB = BLK // L
    BW = 8 * BLK
    ACC = 3 * N_BINS * L
    HALF = SB * BW
    assert n_full % 2 == 0

    mesh = plsc.VectorSubcoreMesh(core_axis_name="c", subcore_axis_name="s")

    @functools.partial(
        pl.kernel,
        mesh=mesh,
        out_type=jax.ShapeDtypeStruct((W, ACC), jnp.float32),
        scratch_types=[
            pltpu.VMEM((2 * NB * BW,), jnp.float32),
            pltpu.VMEM((2 * NB * BW,), jnp.float32),
            pltpu.VMEM((NB * BLK,), jnp.int32),
            pltpu.VMEM((NB * BLK,), jnp.int32),
            pltpu.VMEM((ACC,), jnp.float32),
            pltpu.SemaphoreType.DMA,
            pltpu.SemaphoreType.DMA,
        ],
        compiler_params=pltpu.CompilerParams(needs_layout_passes=False),
    )
    def ece_k(flat_hbm, labels_hbm, out_hbm, buf0, buf1, lbuf0, lbuf1,
              acc, sem0, sem1):
        bufs, lbufs, sems = [buf0, buf1], [lbuf0, lbuf1], [sem0, sem1]
        wid = lax.axis_index("s") * NCORES + lax.axis_index("c")
        lane = lax.iota(jnp.int32, L)
        ones = jnp.ones((L,), jnp.float32)
        for k in range(3 * N_BINS):
            acc[pl.ds(k * L, L)] = jnp.zeros((L,), jnp.float32)

        b0 = base * wid + jnp.minimum(wid, rem)

        HG = GPB // 2

        def block_max(buf, boff, halfw):
            ms = []
            for t in range(HG):
                off = boff + t * L
                vs = ([buf[pl.ds(off + c * BLK, L)] for c in range(8)]
                      + [buf[pl.ds(halfw + off + c * BLK, L)]
                         for c in range(8)])
                while len(vs) > 1:
                    vs = [jnp.maximum(vs[i], vs[i + 1])
                          for i in range(0, len(vs) - 1, 2)]
                ms.append(vs[0])
            return tuple(ms)

        def block_finish(buf, lbuf, boff, goff, halfw, ms):
            exps = [jnp.exp(-m) for m in ms]
            confs = [1.0 / (1.0 + e) for e in exps]
            labs = [lbuf[pl.ds(goff + t * L, L)] for t in range(HG)]
            vlabs = []
            for t in range(HG):
                labidx = ((boff + t * L + lane) + (labs[t] >> 3) * halfw
                          + (labs[t] & 7) * BLK)
                vlabs.append(plsc.load_gather(buf, [labidx]))
            for t in range(HG):
                accv = jnp.where(vlabs[t] == ms[t], 1.0, 0.0)
                accv = accv.astype(jnp.float32)
                bi = jnp.minimum((confs[t] * N_BINS).astype(jnp.int32),
                                 N_BINS - 1)
                fidx = bi * L + lane
                plsc.addupdate_scatter(acc, [fidx], ones)
                plsc.addupdate_scatter(acc, [fidx + (N_BINS * L)], confs[t])
                plsc.addupdate_scatter(acc, [fidx + (2 * N_BINS * L)], accv)

        def start(b, nb, slot):
            h0 = pltpu.async_copy(
                flat_hbm.at[pl.ds(b * BW, nb * BW)],
                bufs[slot].at[pl.ds(0, nb * BW)], sems[slot])
            h1 = pltpu.async_copy(
                flat_hbm.at[pl.ds(HALF + b * BW, nb * BW)],
                bufs[slot].at[pl.ds(nb * BW, nb * BW)], sems[slot])
            h2 = pltpu.async_copy(
                labels_hbm.at[pl.ds(b * BLK, nb * BLK)],
                lbufs[slot].at[pl.ds(0, nb * BLK)], sems[slot])
            return h0, h1, h2

        def drain(b, nb, slot):
            pltpu.make_async_copy(
                flat_hbm.at[pl.ds(b * BW, nb * BW)],
                bufs[slot].at[pl.ds(0, nb * BW)], sems[slot]).wait()
            pltpu.make_async_copy(
                flat_hbm.at[pl.ds(HALF + b * BW, nb * BW)],
                bufs[slot].at[pl.ds(nb * BW, nb * BW)], sems[slot]).wait()
            pltpu.make_async_copy(
                labels_hbm.at[pl.ds(b * BLK, nb * BLK)],
                lbufs[slot].at[pl.ds(0, nb * BLK)], sems[slot]).wait()

        def compute_chunk(slot, nb):
            halfw = nb * BW
            buf, lbuf = bufs[slot], lbufs[slot]
            HL = BLK // 2

            def hb_off(jh):
                return (jh >> 1) * BW + (jh & 1) * HL

            def blk_body(jh, ms_prev):
                ms_new = block_max(buf, hb_off(jh), halfw)
                block_finish(buf, lbuf, hb_off(jh - 1), (jh - 1) * HL,
                             halfw, ms_prev)
                return ms_new

            ms0 = block_max(buf, 0, halfw)
            ms_last = lax.fori_loop(1, 2 * nb, blk_body, ms0)
            block_finish(buf, lbuf, hb_off(2 * nb - 1), (2 * nb - 1) * HL,
                         halfw, ms_last)

        start(b0, NB, 0)
        start(b0 + NB, NB, 1)

        def pair_body(j, carry):
            c0 = 2 * j
            bb = b0 + c0 * NB
            drain(bb, NB, 0)
            compute_chunk(0, NB)

            @pl.when(c0 + 2 < n_full)
            def _():
                start(bb + 2 * NB, NB, 0)

            drain(bb + NB, NB, 1)
            compute_chunk(1, NB)

            @pl.when(c0 + 3 < n_full)
            def _():
                start(bb + 3 * NB, NB, 1)

            return carry

        lax.fori_loop(0, n_full // 2, pair_body, 0)

        if NB_TAIL:
            bt = b0 + n_full * NB
            start(bt, NB_TAIL, 0)
            drain(bt, NB_TAIL, 0)
            compute_chunk(0, NB_TAIL)

        @pl.when(wid < rem)
        def _():
            bx = b0 + base
            start(bx, 1, 1)
            drain(bx, 1, 1)
            compute_chunk(1, 1)

        pltpu.sync_copy(acc, out_hbm.at[wid])

    return ece_k, W, L


def kernel(logits, labels):
    N, C = logits.shape
    part_fn, W, L = _make_partials(N, C)
    flat = logits.T.reshape(2, 8, N // BLK, BLK).transpose(0, 2, 1, 3).reshape(-1)
    parts = part_fn(flat, labels.astype(jnp.int32))
    tot = parts.sum(axis=0).reshape(3, N_BINS, L).sum(axis=-1)
    count, sconf, sacc = tot[0], tot[1], tot[2]
    prop = count / N
    safe = jnp.maximum(count, 1.0)
    diff = jnp.abs(sconf / safe - sacc / safe)
    ece = jnp.sum(jnp.where(count > 0, diff * prop, 0.0), dtype=jnp.float32)
    return ece.reshape(1)


# speedup vs baseline: 2.0744x; 1.0007x over previous
import functools

import jax
import jax.numpy as jnp
from jax import lax
from jax.experimental import pallas as pl
from jax.experimental.pallas import tpu as pltpu
from jax.experimental.pallas import tpu_sc as plsc

N_BINS = 10
BLK = 128


def _make_partials(N, C):
    info = plsc.get_sparse_core_info()
    NCORES, NSUB, L = info.num_cores, info.num_subcores, info.num_lanes
    W = NCORES * NSUB
    assert N % BLK == 0 and C == 16 and L == 16
    SB = N // BLK
    base = SB // W
    rem = SB % W
    NB = 24
    n_full = base // NB
    NB_TAIL = base - n_full * NB
    GPB = BLK // L
    BW = 8 * BLK
    ACC = 3 * N_BINS * L
    HALF = SB * BW
    HALFW = NB * BW
    assert n_full % 2 == 0

    mesh = plsc.VectorSubcoreMesh(core_axis_name="c", subcore_axis_name="s")

    @functools.partial(
        pl.kernel,
        mesh=mesh,
        out_type=jax.ShapeDtypeStruct((W, ACC), jnp.float32),
        scratch_types=[
            pltpu.VMEM((2 * NB * BW,), jnp.float32),
            pltpu.VMEM((2 * NB * BW,), jnp.float32),
            pltpu.VMEM((NB * BLK,), jnp.int32),
            pltpu.VMEM((NB * BLK,), jnp.int32),
            pltpu.VMEM((ACC,), jnp.float32),
            pltpu.SemaphoreType.DMA,
            pltpu.SemaphoreType.DMA,
        ],
        compiler_params=pltpu.CompilerParams(needs_layout_passes=False),
    )
    def ece_k(flat_hbm, labels_hbm, out_hbm, buf0, buf1, lbuf0, lbuf1,
              acc, sem0, sem1):
        bufs, lbufs, sems = [buf0, buf1], [lbuf0, lbuf1], [sem0, sem1]
        wid = lax.axis_index("s") * NCORES + lax.axis_index("c")
        lane = lax.iota(jnp.int32, L)
        ones = jnp.ones((L,), jnp.float32)
        for k in range(3 * N_BINS):
            acc[pl.ds(k * L, L)] = jnp.zeros((L,), jnp.float32)

        b0 = base * wid + jnp.minimum(wid, rem)

        HG = GPB // 2

        def block_max(buf, boff, halfw):
            ms = []
            for t in range(HG):
                off = boff + t * L
                vs = ([buf[pl.ds(off + c * BLK, L)] for c in range(8)]
                      + [buf[pl.ds(halfw + off + c * BLK, L)]
                         for c in range(8)])
                while len(vs) > 1:
                    vs = [jnp.maximum(vs[i], vs[i + 1])
                          for i in range(0, len(vs) - 1, 2)]
                ms.append(vs[0])
            return tuple(ms)

        def block_finish(buf, lbuf, boff, goff, halfw, ms):
            exps = [jnp.exp(-m) for m in ms]
            confs = [1.0 / (1.0 + e) for e in exps]
            labs = [lbuf[pl.ds(goff + t * L, L)] for t in range(HG)]
            vlabs = []
            for t in range(HG):
                labidx = ((boff + t * L + lane) + (labs[t] >> 3) * halfw
                          + (labs[t] & 7) * BLK)
                vlabs.append(plsc.load_gather(buf, [labidx]))
            for t in range(HG):
                accv = jnp.where(vlabs[t] == ms[t], 1.0, 0.0)
                accv = accv.astype(jnp.float32)
                bi = jnp.minimum((confs[t] * N_BINS).astype(jnp.int32),
                                 N_BINS - 1)
                fidx = bi * L + lane
                plsc.addupdate_scatter(acc, [fidx], ones)
                plsc.addupdate_scatter(acc, [fidx + (N_BINS * L)], confs[t])
                plsc.addupdate_scatter(acc, [fidx + (2 * N_BINS * L)], accv)

        def start(b, nb, slot):
            h0 = pltpu.async_copy(
                flat_hbm.at[pl.ds(b * BW, nb * BW)],
                bufs[slot].at[pl.ds(0, nb * BW)], sems[slot])
            h1 = pltpu.async_copy(
                flat_hbm.at[pl.ds(HALF + b * BW, nb * BW)],
                bufs[slot].at[pl.ds(HALFW, nb * BW)], sems[slot])
            h2 = pltpu.async_copy(
                labels_hbm.at[pl.ds(b * BLK, nb * BLK)],
                lbufs[slot].at[pl.ds(0, nb * BLK)], sems[slot])
            return h0, h1, h2

        def drain(b, nb, slot):
            pltpu.make_async_copy(
                flat_hbm.at[pl.ds(b * BW, nb * BW)],
                bufs[slot].at[pl.ds(0, nb * BW)], sems[slot]).wait()
            pltpu.make_async_copy(
                flat_hbm.at[pl.ds(HALF + b * BW, nb * BW)],
                bufs[slot].at[pl.ds(HALFW, nb * BW)], sems[slot]).wait()
            pltpu.make_async_copy(
                labels_hbm.at[pl.ds(b * BLK, nb * BLK)],
                lbufs[slot].at[pl.ds(0, nb * BLK)], sems[slot]).wait()

        def compute_chunk(slot, nb):
            halfw = HALFW
            buf, lbuf = bufs[slot], lbufs[slot]
            HL = BLK // 2

            def hb_off(jh):
                return (jh >> 1) * BW + (jh & 1) * HL

            def blk_body(jh, ms_prev):
                ms_new = block_max(buf, hb_off(jh), halfw)
                block_finish(buf, lbuf, hb_off(jh - 1), (jh - 1) * HL,
                             halfw, ms_prev)
                return ms_new

            ms0 = block_max(buf, 0, halfw)
            ms_last = lax.fori_loop(1, 2 * nb, blk_body, ms0)
            block_finish(buf, lbuf, hb_off(2 * nb - 1), (2 * nb - 1) * HL,
                         halfw, ms_last)

        start(b0, NB, 0)
        start(b0 + NB, NB, 1)

        def pair_body(j, carry):
            c0 = 2 * j
            bb = b0 + c0 * NB
            drain(bb, NB, 0)
            compute_chunk(0, NB)

            @pl.when(c0 + 2 < n_full)
            def _():
                start(bb + 2 * NB, NB, 0)

            drain(bb + NB, NB, 1)
            compute_chunk(1, NB)

            @pl.when(c0 + 3 < n_full)
            def _():
                start(bb + 3 * NB, NB, 1)

            return carry

        lax.fori_loop(0, n_full // 2, pair_body, 0)

        if NB_TAIL:
            bt = b0 + n_full * NB
            start(bt, NB_TAIL, 0)
            drain(bt, NB_TAIL, 0)
            compute_chunk(0, NB_TAIL)

        @pl.when(wid < rem)
        def _():
            bx = b0 + base
            start(bx, 1, 1)
            drain(bx, 1, 1)
            compute_chunk(1, 1)

        pltpu.sync_copy(acc, out_hbm.at[wid])

    return ece_k, W, L


def kernel(logits, labels):
    N, C = logits.shape
    part_fn, W, L = _make_partials(N, C)
    flat = logits.T.reshape(2, 8, N // BLK, BLK).transpose(0, 2, 1, 3).reshape(-1)
    parts = part_fn(flat, labels.astype(jnp.int32))
    tot = parts.sum(axis=0).reshape(3, N_BINS, L).sum(axis=-1)
    count, sconf, sacc = tot[0], tot[1], tot[2]
    prop = count / N
    safe = jnp.maximum(count, 1.0)
    diff = jnp.abs(sconf / safe - sacc / safe)
    ece = jnp.sum(jnp.where(count > 0, diff * prop, 0.0), dtype=jnp.float32)
    return ece.reshape(1)
